# Initial kernel scaffold; baseline (speedup 1.0000x reference)
import functools

import jax
import jax.numpy as jnp
from jax import lax
from jax.experimental import pallas as pl
from jax.experimental.pallas import tpu as pltpu, tpu_sc as plsc

HID = 128

_NC = 2
_NS = 16
_NW = _NC * _NS
_CHUNK = 256


def _silu(x):
    return x * (1.0 / (1.0 + jnp.exp(-x)))


def _sig(x):
    return 1.0 / (1.0 + jnp.exp(-x))


def _sc_gather(hidden, coords16, rowp, colp, epad):
    chunks = epad // (_NW * _CHUNK)
    mesh = plsc.VectorSubcoreMesh(core_axis_name="c", subcore_axis_name="s")
    out_type = (
        jax.ShapeDtypeStruct((epad, HID), jnp.float32),
        jax.ShapeDtypeStruct((epad, HID), jnp.float32),
        jax.ShapeDtypeStruct((epad, 16), jnp.float32),
        jax.ShapeDtypeStruct((epad, 16), jnp.float32),
    )

    @functools.partial(
        pl.kernel, mesh=mesh, out_type=out_type,
        scratch_types=[
            pltpu.VMEM((_CHUNK,), jnp.int32),
            pltpu.VMEM((_CHUNK,), jnp.int32),
            pltpu.VMEM((_CHUNK, HID), jnp.float32),
            pltpu.VMEM((_CHUNK, 16), jnp.float32),
            pltpu.SemaphoreType.DMA,
        ],
    )
    def k(hid_hbm, c16_hbm, row_hbm, col_hbm, hr_hbm, hc_hbm, cr_hbm, cc_hbm,
          idxr, idxc, hbuf, cbuf, sem):
        wid = lax.axis_index("s") * _NC + lax.axis_index("c")

        def body(j, carry):
            base = (wid * chunks + j) * _CHUNK
            sl = pl.ds(base, _CHUNK)
            pltpu.sync_copy(row_hbm.at[sl], idxr)
            pltpu.sync_copy(col_hbm.at[sl], idxc)
            pltpu.async_copy(hid_hbm.at[idxr], hbuf, sem).wait()
            pltpu.sync_copy(hbuf, hr_hbm.at[sl])
            pltpu.async_copy(hid_hbm.at[idxc], hbuf, sem).wait()
            pltpu.sync_copy(hbuf, hc_hbm.at[sl])
            pltpu.async_copy(c16_hbm.at[idxr], cbuf, sem).wait()
            pltpu.sync_copy(cbuf, cr_hbm.at[sl])
            pltpu.async_copy(c16_hbm.at[idxc], cbuf, sem).wait()
            pltpu.sync_copy(cbuf, cc_hbm.at[sl])
            return carry

        lax.fori_loop(0, chunks, body, 0)

    return k(hidden, coords16, rowp, colp)


def _sc_scatter(m, aux, rowp, z128, z16, n, epad):
    chunks = epad // (_NW * _CHUNK)
    rpt = n // _NS
    mesh = plsc.VectorSubcoreMesh(core_axis_name="c", subcore_axis_name="s")
    out_type = (
        jax.ShapeDtypeStruct((2 * n, HID), jnp.float32),
        jax.ShapeDtypeStruct((2 * n, 16), jnp.float32),
    )

    @functools.partial(
        pl.kernel, mesh=mesh, out_type=out_type,
        scratch_types=[
            pltpu.VMEM((_CHUNK,), jnp.int32),
            pltpu.VMEM((_CHUNK, HID), jnp.float32),
            pltpu.VMEM((_CHUNK, 16), jnp.float32),
            pltpu.VMEM_SHARED((n, HID), jnp.float32),
            pltpu.VMEM_SHARED((n, 16), jnp.float32),
        ],
    )
    def k(m_hbm, a_hbm, row_hbm, z128_hbm, z16_hbm, o128_hbm, o16_hbm,
          idxb, mbuf, abuf, acc128, acc16):
        c = lax.axis_index("c")
        s = lax.axis_index("s")
        rsl = pl.ds(s * rpt, rpt)
        pltpu.sync_copy(z128_hbm.at[rsl], acc128.at[rsl])
        pltpu.sync_copy(z16_hbm.at[rsl], acc16.at[rsl])
        plsc.subcore_barrier()
        wid = c * _NS + s

        def body(j, carry):
            base = (wid * chunks + j) * _CHUNK
            sl = pl.ds(base, _CHUNK)
            pltpu.sync_copy(row_hbm.at[sl], idxb)
            pltpu.sync_copy(m_hbm.at[sl], mbuf)
            pltpu.sync_copy(a_hbm.at[sl], abuf)
            pltpu.sync_copy(mbuf, acc128.at[idxb], add=True)
            pltpu.sync_copy(abuf, acc16.at[idxb], add=True)
            return carry

        lax.fori_loop(0, chunks, body, 0)
        plsc.subcore_barrier()
        osl = pl.ds(c * n + s * rpt, rpt)
        pltpu.sync_copy(acc128.at[rsl], o128_hbm.at[osl])
        pltpu.sync_copy(acc16.at[rsl], o16_hbm.at[osl])

    return k(m, aux, rowp, z128, z16)


def _tc_edge(hr, hc, cr, cc, w1a, w1b, w1c, w2, wc1, smalls, e_real, epad):
    BE = 1024
    nb = epad // BE

    def body(hr_ref, hc_ref, cr_ref, cc_ref, w1a_ref, w1b_ref, w1c_ref,
             w2_ref, wc1_ref, sm_ref, m_ref, aux_ref):
        sm = sm_ref[...]
        rel = cr_ref[...] - cc_ref[...]
        d2 = jnp.sum(rel * rel, axis=1, keepdims=True)
        t = jnp.dot(hr_ref[...], w1a_ref[...], preferred_element_type=jnp.float32)
        t = t + jnp.dot(hc_ref[...], w1b_ref[...], preferred_element_type=jnp.float32)
        t = t + d2 * w1c_ref[...] + sm[0:1]
        m = _silu(t)
        t2 = jnp.dot(m, w2_ref[...], preferred_element_type=jnp.float32) + sm[1:2]
        m = _silu(t2)
        ba = jnp.sum(sm[3:4] * (1.0 / HID), axis=1, keepdims=True)
        att = _sig(jnp.sum(m * sm[2:3], axis=1, keepdims=True) + ba)
        m = m * att
        t3 = jnp.dot(m, wc1_ref[...], preferred_element_type=jnp.float32) + sm[4:5]
        cwh = _silu(t3)
        cw = jnp.sum(cwh * sm[5:6], axis=1, keepdims=True)
        rowid = (jax.lax.broadcasted_iota(jnp.int32, (BE, 1), 0)
                 + pl.program_id(0) * BE)
        msk = (rowid < e_real).astype(jnp.float32)
        m_ref[...] = m * msk
        aux = rel * cw
        lane = jax.lax.broadcasted_iota(jnp.int32, (BE, 16), 1)
        aux = jnp.where(lane == 3, 1.0, aux)
        aux_ref[...] = aux * msk

    const2 = lambda i: (0, 0)
    return pl.pallas_call(
        body,
        grid=(nb,),
        in_specs=[
            pl.BlockSpec((BE, HID), lambda i: (i, 0)),
            pl.BlockSpec((BE, HID), lambda i: (i, 0)),
            pl.BlockSpec((BE, 16), lambda i: (i, 0)),
            pl.BlockSpec((BE, 16), lambda i: (i, 0)),
            pl.BlockSpec((HID, HID), const2),
            pl.BlockSpec((HID, HID), const2),
            pl.BlockSpec((1, HID), const2),
            pl.BlockSpec((HID, HID), const2),
            pl.BlockSpec((HID, HID), const2),
            pl.BlockSpec((8, HID), const2),
        ],
        out_specs=[
            pl.BlockSpec((BE, HID), lambda i: (i, 0)),
            pl.BlockSpec((BE, 16), lambda i: (i, 0)),
        ],
        out_shape=[
            jax.ShapeDtypeStruct((epad, HID), jnp.float32),
            jax.ShapeDtypeStruct((epad, 16), jnp.float32),
        ],
    )(hr, hc, cr, cc, w1a, w1b, w1c, w2, wc1, smalls)


def _tc_node(p128, p16, hidden, coords16, wn1a, wn1b, wn2, smalls, n):
    BN = 1000
    nb = n // BN

    def body(a0_ref, a1_ref, x0_ref, x1_ref, h_ref, c_ref,
             wa_ref, wb_ref, w2_ref, sm_ref, hpre_ref, oc_ref, st_ref, acc):
        i = pl.program_id(0)
        sm = sm_ref[...]
        hagg = a0_ref[...] + a1_ref[...]
        aux = x0_ref[...] + x1_ref[...]
        h = h_ref[...]
        t = (jnp.dot(h, wa_ref[...], preferred_element_type=jnp.float32)
             + jnp.dot(hagg, wb_ref[...], preferred_element_type=jnp.float32)
             + sm[0:1])
        hu = jnp.dot(_silu(t), w2_ref[...], preferred_element_type=jnp.float32) + sm[1:2]
        hpre = h + hu
        hpre_ref[...] = hpre

        @pl.when(i == 0)
        def _():
            acc[...] = jnp.zeros_like(acc)

        acc[0:1, :] = acc[0:1, :] + jnp.sum(hpre, axis=0, keepdims=True)
        acc[1:2, :] = acc[1:2, :] + jnp.sum(hpre * hpre, axis=0, keepdims=True)
        st_ref[...] = acc[...]

        lane = jax.lax.broadcasted_iota(jnp.int32, (BN, 16), 1)
        deg = jnp.sum(jnp.where(lane == 3, aux, 0.0), axis=1, keepdims=True)
        degc = jnp.maximum(deg, 1.0)
        cagg = jnp.where(lane < 3, aux, 0.0)
        oc_ref[...] = c_ref[...] + cagg / degc

    const2 = lambda i: (0, 0)
    return pl.pallas_call(
        body,
        grid=(nb,),
        in_specs=[
            pl.BlockSpec((BN, HID), lambda i: (i, 0)),
            pl.BlockSpec((BN, HID), lambda i: (i + nb, 0)),
            pl.BlockSpec((BN, 16), lambda i: (i, 0)),
            pl.BlockSpec((BN, 16), lambda i: (i + nb, 0)),
            pl.BlockSpec((BN, HID), lambda i: (i, 0)),
            pl.BlockSpec((BN, 16), lambda i: (i, 0)),
            pl.BlockSpec((HID, HID), const2),
            pl.BlockSpec((HID, HID), const2),
            pl.BlockSpec((HID, HID), const2),
            pl.BlockSpec((8, HID), const2),
        ],
        out_specs=[
            pl.BlockSpec((BN, HID), lambda i: (i, 0)),
            pl.BlockSpec((BN, 16), lambda i: (i, 0)),
            pl.BlockSpec((8, HID), const2),
        ],
        out_shape=[
            jax.ShapeDtypeStruct((n, HID), jnp.float32),
            jax.ShapeDtypeStruct((n, 16), jnp.float32),
            jax.ShapeDtypeStruct((8, HID), jnp.float32),
        ],
        scratch_shapes=[pltpu.VMEM((8, HID), jnp.float32)],
    )(p128, p16, hidden, coords16, wn1a, wn1b, wn2, smalls)


def _tc_norm(hpre, stats, n):
    BN = 1000
    nb = n // BN

    def body(h_ref, st_ref, o_ref):
        s = st_ref[...]
        mu = s[0:1, :] * (1.0 / n)
        var = (jnp.sum(s[1:2, :], axis=1, keepdims=True) * (1.0 / n)
               - jnp.sum(mu * mu, axis=1, keepdims=True))
        r = jax.lax.rsqrt(1e-6 + var)
        o_ref[...] = (h_ref[...] - mu) * r

    return pl.pallas_call(
        body,
        grid=(nb,),
        in_specs=[
            pl.BlockSpec((BN, HID), lambda i: (i, 0)),
            pl.BlockSpec((8, HID), lambda i: (0, 0)),
        ],
        out_specs=pl.BlockSpec((BN, HID), lambda i: (i, 0)),
        out_shape=jax.ShapeDtypeStruct((n, HID), jnp.float32),
    )(hpre, stats)


def kernel(batch_coords, batch_hidden, edges, We1, be1, We2, be2, Wa, ba,
           Wc1, bc1, Wc2, Wn1, bn1, Wn2, bn2):
    f32 = jnp.float32
    n = batch_hidden.shape[0]
    e = edges.shape[1]
    tile_e = _NW * _CHUNK
    epad = ((e + tile_e - 1) // tile_e) * tile_e

    coords16 = jnp.zeros((n, 16), f32).at[:, :3].set(batch_coords)
    pad = epad - e
    rowp = jnp.concatenate([edges[0], jnp.zeros((pad,), jnp.int32)])
    colp = jnp.concatenate([edges[1], jnp.zeros((pad,), jnp.int32)])

    hr, hc, cr, cc = _sc_gather(batch_hidden, coords16, rowp, colp, epad)

    w1a = We1[:HID]
    w1b = We1[HID:2 * HID]
    w1c = We1[2 * HID:]
    smalls = (jnp.zeros((8, HID), f32)
              .at[0].set(be1).at[1].set(be2).at[2].set(Wa[:, 0])
              .at[3].set(jnp.full((HID,), ba[0], f32))
              .at[4].set(bc1).at[5].set(Wc2[:, 0]))
    m, aux = _tc_edge(hr, hc, cr, cc, w1a, w1b, w1c, We2, Wc1, smalls, e, epad)

    z128 = jnp.zeros((n, HID), f32)
    z16 = jnp.zeros((n, 16), f32)
    o128, o16 = _sc_scatter(m, aux, rowp, z128, z16, n, epad)

    wn1a = Wn1[:HID]
    wn1b = Wn1[HID:]
    smalls2 = jnp.zeros((8, HID), f32).at[0].set(bn1).at[1].set(bn2)
    hpre, oc16, stats = _tc_node(o128, o16, batch_hidden, coords16,
                                 wn1a, wn1b, Wn2, smalls2, n)
    out_hidden = _tc_norm(hpre, stats, n)
    return (oc16[:, :3], out_hidden)


# speedup vs baseline: 2.6398x; 2.6398x over previous
import functools

import jax
import jax.numpy as jnp
from jax import lax
from jax.experimental import pallas as pl
from jax.experimental.pallas import tpu as pltpu, tpu_sc as plsc

HID = 128

_NC = 2
_NS = 16
_NW = _NC * _NS
_CHUNK = 128


def _silu(x):
    return x * (1.0 / (1.0 + jnp.exp(-x)))


def _sig(x):
    return 1.0 / (1.0 + jnp.exp(-x))


def _sc_gather(hidden, coords4, rowp, colp, n, epad):
    chunks = epad // (_NW * _CHUNK)
    mesh = plsc.VectorSubcoreMesh(core_axis_name="c", subcore_axis_name="s")
    out_type = (
        jax.ShapeDtypeStruct((epad, HID), jnp.float32),
        jax.ShapeDtypeStruct((epad, HID), jnp.float32),
        jax.ShapeDtypeStruct((epad * 16,), jnp.float32),
    )

    @functools.partial(
        pl.kernel, mesh=mesh, out_type=out_type,
        scratch_types=[
            pltpu.VMEM((_CHUNK,), jnp.int32),
            pltpu.VMEM((_CHUNK,), jnp.int32),
            pltpu.VMEM((_CHUNK, HID), jnp.float32),
            pltpu.VMEM((_CHUNK, HID), jnp.float32),
            pltpu.VMEM((_CHUNK * 16,), jnp.float32),
            pltpu.VMEM((n * 4,), jnp.float32),
            pltpu.SemaphoreType.DMA,
        ],
        compiler_params=pltpu.CompilerParams(needs_layout_passes=False),
    )
    def k(hid_hbm, c4_hbm, row_hbm, col_hbm, hr_hbm, hc_hbm, rl_hbm,
          idxr, idxc, hbufr, hbufc, rbuf, ctab, sem):
        wid = lax.axis_index("s") * _NC + lax.axis_index("c")
        pltpu.sync_copy(c4_hbm, ctab)

        def zero(i, carry):
            rbuf[pl.ds(i * 16, 16)] = jnp.zeros((16,), jnp.float32)
            return carry

        lax.fori_loop(0, _CHUNK, zero, 0)

        def body(j, carry):
            base = (wid * chunks + j) * _CHUNK
            sl = pl.ds(base, _CHUNK)
            pltpu.sync_copy(row_hbm.at[sl], idxr)
            pltpu.sync_copy(col_hbm.at[sl], idxc)
            cpr = pltpu.async_copy(hid_hbm.at[idxr], hbufr, sem)
            cpc = pltpu.async_copy(hid_hbm.at[idxc], hbufc, sem)
            for g in range(_CHUNK // 16):
                ir4 = idxr[pl.ds(g * 16, 16)] * 4
                ic4 = idxc[pl.ds(g * 16, 16)] * 4
                eid = lax.iota(jnp.int32, 16) * 16 + (g * 16 * 16)
                d2 = jnp.zeros((16,), jnp.float32)
                for comp in range(3):
                    rk = (plsc.load_gather(ctab, [ir4 + comp])
                          - plsc.load_gather(ctab, [ic4 + comp]))
                    d2 = d2 + rk * rk
                    plsc.store_scatter(rbuf, [eid + comp], rk)
                plsc.store_scatter(rbuf, [eid + 3], d2)
            pltpu.sync_copy(rbuf, rl_hbm.at[pl.ds(base * 16, _CHUNK * 16)])
            cpr.wait()
            cpc.wait()
            pltpu.sync_copy(hbufr, hr_hbm.at[sl])
            pltpu.sync_copy(hbufc, hc_hbm.at[sl])
            return carry

        lax.fori_loop(0, chunks, body, 0)

    return k(hidden, coords4, rowp, colp)


def _sc_scatter(m, aux128, rowp, z128, npad, epad):
    chunks = epad // (_NW * _CHUNK)
    rpt = npad // _NS
    mesh = plsc.VectorSubcoreMesh(core_axis_name="c", subcore_axis_name="s")
    out_type = (
        jax.ShapeDtypeStruct((2 * npad, HID), jnp.float32),
        jax.ShapeDtypeStruct((2 * npad, HID), jnp.float32),
    )
    pieces = []
    o = 0
    while o < rpt:
        ln = min(_CHUNK, rpt - o)
        pieces.append((o, ln))
        o += ln

    @functools.partial(
        pl.kernel, mesh=mesh, out_type=out_type,
        scratch_types=[
            pltpu.VMEM((_CHUNK,), jnp.int32),
            pltpu.VMEM((_CHUNK, HID), jnp.float32),
            pltpu.VMEM_SHARED((npad, HID), jnp.float32),
        ],
    )
    def k(m_hbm, a_hbm, row_hbm, z128_hbm, o128_hbm, oa_hbm,
          idxb, mbuf, acc128):
        c = lax.axis_index("c")
        s = lax.axis_index("s")
        wid = c * _NS + s

        def zero_acc():
            pltpu.sync_copy(z128_hbm, mbuf)
            for o_, ln in pieces:
                pltpu.sync_copy(mbuf.at[pl.ds(0, ln)],
                                acc128.at[pl.ds(s * rpt + o_, ln)])

        def drain_acc(dst):
            for o_, ln in pieces:
                pltpu.sync_copy(acc128.at[pl.ds(s * rpt + o_, ln)],
                                mbuf.at[pl.ds(0, ln)])
                pltpu.sync_copy(mbuf.at[pl.ds(0, ln)],
                                dst.at[pl.ds(c * npad + s * rpt + o_, ln)])

        def scat_loop(src_hbm):
            def body(j, carry):
                base = (wid * chunks + j) * _CHUNK
                sl = pl.ds(base, _CHUNK)
                pltpu.sync_copy(row_hbm.at[sl], idxb)
                pltpu.sync_copy(src_hbm.at[sl], mbuf)
                pltpu.sync_copy(mbuf, acc128.at[idxb], add=True)
                return carry
            lax.fori_loop(0, chunks, body, 0)

        zero_acc()
        plsc.subcore_barrier()
        scat_loop(m_hbm)
        plsc.subcore_barrier()
        drain_acc(o128_hbm)
        plsc.subcore_barrier()
        zero_acc()
        plsc.subcore_barrier()
        scat_loop(a_hbm)
        plsc.subcore_barrier()
        drain_acc(oa_hbm)

    return k(m, aux128, rowp, z128)


def _tc_edge(hr, hc, rels, w1a, w1b, w1c, w2, wc1, smalls, e_real, epad):
    BE = 1024
    nb = epad // BE

    def body(hr_ref, hc_ref, rl_ref, w1a_ref, w1b_ref, w1c_ref,
             w2_ref, wc1_ref, sm_ref, m_ref, aux_ref):
        sm = sm_ref[...]
        lane = jax.lax.broadcasted_iota(jnp.int32, (BE, 16), 1)
        rl = rl_ref[...]
        rel = jnp.where(lane < 3, rl, 0.0)
        d2 = jnp.sum(jnp.where(lane == 3, rl, 0.0), axis=1, keepdims=True)
        t = jnp.dot(hr_ref[...], w1a_ref[...], preferred_element_type=jnp.float32)
        t = t + jnp.dot(hc_ref[...], w1b_ref[...], preferred_element_type=jnp.float32)
        t = t + d2 * w1c_ref[...] + sm[0:1]
        m = _silu(t)
        t2 = jnp.dot(m, w2_ref[...], preferred_element_type=jnp.float32) + sm[1:2]
        m = _silu(t2)
        ba = jnp.sum(sm[3:4] * (1.0 / HID), axis=1, keepdims=True)
        att = _sig(jnp.sum(m * sm[2:3], axis=1, keepdims=True) + ba)
        m = m * att
        t3 = jnp.dot(m, wc1_ref[...], preferred_element_type=jnp.float32) + sm[4:5]
        cwh = _silu(t3)
        cw = jnp.sum(cwh * sm[5:6], axis=1, keepdims=True)
        rowid = (jax.lax.broadcasted_iota(jnp.int32, (BE, 1), 0)
                 + pl.program_id(0) * BE)
        msk = (rowid < e_real).astype(jnp.float32)
        m_ref[...] = m * msk
        aux = rel * cw
        aux = jnp.where(lane == 3, 1.0, aux)
        aux_ref[...] = jnp.concatenate(
            [aux * msk, jnp.zeros((BE, HID - 16), jnp.float32)], axis=1)

    const2 = lambda i: (0, 0)
    return pl.pallas_call(
        body,
        grid=(nb,),
        in_specs=[
            pl.BlockSpec((BE, HID), lambda i: (i, 0)),
            pl.BlockSpec((BE, HID), lambda i: (i, 0)),
            pl.BlockSpec((BE, 16), lambda i: (i, 0)),
            pl.BlockSpec((HID, HID), const2),
            pl.BlockSpec((HID, HID), const2),
            pl.BlockSpec((1, HID), const2),
            pl.BlockSpec((HID, HID), const2),
            pl.BlockSpec((HID, HID), const2),
            pl.BlockSpec((8, HID), const2),
        ],
        out_specs=[
            pl.BlockSpec((BE, HID), lambda i: (i, 0)),
            pl.BlockSpec((BE, HID), lambda i: (i, 0)),
        ],
        out_shape=[
            jax.ShapeDtypeStruct((epad, HID), jnp.float32),
            jax.ShapeDtypeStruct((epad, HID), jnp.float32),
        ],
    )(hr, hc, rels, w1a, w1b, w1c, w2, wc1, smalls)


def _tc_node(p0, p1, x0, x1, hidden, coords128, wn1a, wn1b, wn2, smalls, n):
    BN = 1000
    nb = n // BN

    def body(a0_ref, a1_ref, x0_ref, x1_ref, h_ref, c_ref,
             wa_ref, wb_ref, w2_ref, sm_ref, hpre_ref, oc_ref, st_ref, acc):
        i = pl.program_id(0)
        sm = sm_ref[...]
        hagg = a0_ref[...] + a1_ref[...]
        aux = x0_ref[...] + x1_ref[...]
        h = h_ref[...]
        t = (jnp.dot(h, wa_ref[...], preferred_element_type=jnp.float32)
             + jnp.dot(hagg, wb_ref[...], preferred_element_type=jnp.float32)
             + sm[0:1])
        hu = jnp.dot(_silu(t), w2_ref[...], preferred_element_type=jnp.float32) + sm[1:2]
        hpre = h + hu
        hpre_ref[...] = hpre

        @pl.when(i == 0)
        def _():
            acc[...] = jnp.zeros_like(acc)

        acc[0:1, :] = acc[0:1, :] + jnp.sum(hpre, axis=0, keepdims=True)
        acc[1:2, :] = acc[1:2, :] + jnp.sum(hpre * hpre, axis=0, keepdims=True)
        st_ref[...] = acc[...]

        lane = jax.lax.broadcasted_iota(jnp.int32, (BN, HID), 1)
        deg = jnp.sum(jnp.where(lane == 3, aux, 0.0), axis=1, keepdims=True)
        degc = jnp.maximum(deg, 1.0)
        cagg = jnp.where(lane < 3, aux, 0.0)
        oc_ref[...] = c_ref[...] + cagg / degc

    const2 = lambda i: (0, 0)
    return pl.pallas_call(
        body,
        grid=(nb,),
        in_specs=[
            pl.BlockSpec((BN, HID), lambda i: (i, 0)),
            pl.BlockSpec((BN, HID), lambda i: (i, 0)),
            pl.BlockSpec((BN, HID), lambda i: (i, 0)),
            pl.BlockSpec((BN, HID), lambda i: (i, 0)),
            pl.BlockSpec((BN, HID), lambda i: (i, 0)),
            pl.BlockSpec((BN, HID), lambda i: (i, 0)),
            pl.BlockSpec((HID, HID), const2),
            pl.BlockSpec((HID, HID), const2),
            pl.BlockSpec((HID, HID), const2),
            pl.BlockSpec((8, HID), const2),
        ],
        out_specs=[
            pl.BlockSpec((BN, HID), lambda i: (i, 0)),
            pl.BlockSpec((BN, HID), lambda i: (i, 0)),
            pl.BlockSpec((8, HID), const2),
        ],
        out_shape=[
            jax.ShapeDtypeStruct((n, HID), jnp.float32),
            jax.ShapeDtypeStruct((n, HID), jnp.float32),
            jax.ShapeDtypeStruct((8, HID), jnp.float32),
        ],
        scratch_shapes=[pltpu.VMEM((8, HID), jnp.float32)],
    )(p0, p1, x0, x1, hidden, coords128, wn1a, wn1b, wn2, smalls)


def _tc_norm(hpre, stats, n):
    BN = 1000
    nb = n // BN

    def body(h_ref, st_ref, o_ref):
        s = st_ref[...]
        mu = s[0:1, :] * (1.0 / n)
        var = (jnp.sum(s[1:2, :], axis=1, keepdims=True) * (1.0 / n)
               - jnp.sum(mu * mu, axis=1, keepdims=True))
        r = jax.lax.rsqrt(1e-6 + var)
        o_ref[...] = (h_ref[...] - mu) * r

    return pl.pallas_call(
        body,
        grid=(nb,),
        in_specs=[
            pl.BlockSpec((BN, HID), lambda i: (i, 0)),
            pl.BlockSpec((8, HID), lambda i: (0, 0)),
        ],
        out_specs=pl.BlockSpec((BN, HID), lambda i: (i, 0)),
        out_shape=jax.ShapeDtypeStruct((n, HID), jnp.float32),
    )(hpre, stats)


def kernel(batch_coords, batch_hidden, edges, We1, be1, We2, be2, Wa, ba,
           Wc1, bc1, Wc2, Wn1, bn1, Wn2, bn2):
    f32 = jnp.float32
    n = batch_hidden.shape[0]
    e = edges.shape[1]
    tile_e = _NW * _CHUNK
    epad = ((e + tile_e - 1) // tile_e) * tile_e

    coords128 = jnp.zeros((n, HID), f32).at[:, :3].set(batch_coords)
    coords4 = jnp.zeros((n, 4), f32).at[:, :3].set(batch_coords).reshape(-1)
    pad = epad - e
    rowp = jnp.concatenate([edges[0], jnp.zeros((pad,), jnp.int32)])
    colp = jnp.concatenate([edges[1], jnp.zeros((pad,), jnp.int32)])

    hr, hc, rels_flat = _sc_gather(batch_hidden, coords4, rowp, colp, n, epad)
    rels = rels_flat.reshape(epad, 16)

    w1a = We1[:HID]
    w1b = We1[HID:2 * HID]
    w1c = We1[2 * HID:]
    smalls = (jnp.zeros((8, HID), f32)
              .at[0].set(be1).at[1].set(be2).at[2].set(Wa[:, 0])
              .at[3].set(jnp.full((HID,), ba[0], f32))
              .at[4].set(bc1).at[5].set(Wc2[:, 0]))
    m, aux = _tc_edge(hr, hc, rels, w1a, w1b, w1c, We2, Wc1, smalls, e, epad)

    npad = ((n + 8 * _NS - 1) // (8 * _NS)) * (8 * _NS)
    z128 = jnp.zeros((_CHUNK, HID), f32)
    o128, oa = _sc_scatter(m, aux, rowp, z128, npad, epad)
    p0, p1 = o128[:n], o128[npad:npad + n]
    x0, x1 = oa[:n], oa[npad:npad + n]

    wn1a = Wn1[:HID]
    wn1b = Wn1[HID:]
    smalls2 = jnp.zeros((8, HID), f32).at[0].set(bn1).at[1].set(bn2)
    hpre, oc, stats = _tc_node(p0, p1, x0, x1, batch_hidden, coords128,
                               wn1a, wn1b, Wn2, smalls2, n)
    out_hidden = _tc_norm(hpre, stats, n)
    return (oc[:, :3], out_hidden)


# speedup vs baseline: 2.7743x; 1.0510x over previous
import functools

import jax
import jax.numpy as jnp
from jax import lax
from jax.experimental import pallas as pl
from jax.experimental.pallas import tpu as pltpu, tpu_sc as plsc

HID = 128

_NC = 2
_NS = 16
_NW = _NC * _NS
_CHUNK = 128


def _silu(x):
    return x * (1.0 / (1.0 + jnp.exp(-x)))


def _sig(x):
    return 1.0 / (1.0 + jnp.exp(-x))


def _sc_gather(hidden, coords4, rowp, colp, n, epad):
    chunks = epad // (_NW * _CHUNK)
    mesh = plsc.VectorSubcoreMesh(core_axis_name="c", subcore_axis_name="s")
    out_type = (
        jax.ShapeDtypeStruct((epad, HID), jnp.float32),
        jax.ShapeDtypeStruct((epad, HID), jnp.float32),
        jax.ShapeDtypeStruct((epad * 16,), jnp.float32),
    )

    @functools.partial(
        pl.kernel, mesh=mesh, out_type=out_type,
        scratch_types=[
            [pltpu.VMEM((_CHUNK,), jnp.int32) for _ in range(2)],
            [pltpu.VMEM((_CHUNK,), jnp.int32) for _ in range(2)],
            [pltpu.VMEM((_CHUNK, HID), jnp.float32) for _ in range(2)],
            [pltpu.VMEM((_CHUNK, HID), jnp.float32) for _ in range(2)],
            [pltpu.VMEM((_CHUNK * 16,), jnp.float32) for _ in range(2)],
            pltpu.VMEM((n * 4,), jnp.float32),
            [pltpu.SemaphoreType.DMA for _ in range(2)],
            pltpu.SemaphoreType.DMA,
        ],
        compiler_params=pltpu.CompilerParams(needs_layout_passes=False),
    )
    def k(hid_hbm, c4_hbm, row_hbm, col_hbm, hr_hbm, hc_hbm, rl_hbm,
          idxr, idxc, hbufr, hbufc, rbuf, ctab, gsem, wsem):
        wid = lax.axis_index("s") * _NC + lax.axis_index("c")
        pltpu.sync_copy(c4_hbm, ctab)

        def zero(i, carry):
            rbuf[0][pl.ds(i * 16, 16)] = jnp.zeros((16,), jnp.float32)
            rbuf[1][pl.ds(i * 16, 16)] = jnp.zeros((16,), jnp.float32)
            return carry

        lax.fori_loop(0, _CHUNK, zero, 0)

        def rel_compute(b):
            for g in range(_CHUNK // 16):
                ir4 = idxr[b][pl.ds(g * 16, 16)] * 4
                ic4 = idxc[b][pl.ds(g * 16, 16)] * 4
                eid = lax.iota(jnp.int32, 16) * 16 + (g * 16 * 16)
                d2 = jnp.zeros((16,), jnp.float32)
                for comp in range(3):
                    rk = (plsc.load_gather(ctab, [ir4 + comp])
                          - plsc.load_gather(ctab, [ic4 + comp]))
                    d2 = d2 + rk * rk
                    plsc.store_scatter(rbuf[b], [eid + comp], rk)
                plsc.store_scatter(rbuf[b], [eid + 3], d2)

        def body(j, carry):
            gd = [None, None]
            for b in range(2):
                base = (wid * chunks + 2 * j + b) * _CHUNK
                sl = pl.ds(base, _CHUNK)
                pltpu.sync_copy(row_hbm.at[sl], idxr[b])
                pltpu.sync_copy(col_hbm.at[sl], idxc[b])
                gd[b] = (pltpu.async_copy(hid_hbm.at[idxr[b]], hbufr[b], gsem[b]),
                         pltpu.async_copy(hid_hbm.at[idxc[b]], hbufc[b], gsem[b]))
            wd = []
            for b in range(2):
                base = (wid * chunks + 2 * j + b) * _CHUNK
                sl = pl.ds(base, _CHUNK)
                rel_compute(b)
                wd.append(pltpu.async_copy(
                    rbuf[b], rl_hbm.at[pl.ds(base * 16, _CHUNK * 16)], wsem))
                gd[b][0].wait()
                gd[b][1].wait()
                wd.append(pltpu.async_copy(hbufr[b], hr_hbm.at[sl], wsem))
                wd.append(pltpu.async_copy(hbufc[b], hc_hbm.at[sl], wsem))
            for d in wd:
                d.wait()
            return carry

        lax.fori_loop(0, chunks // 2, body, 0)

    return k(hidden, coords4, rowp, colp)


def _sc_scatter(m, aux128, rowp, z128, npad, epad):
    chunks = epad // (_NW * _CHUNK)
    rpt = npad // _NS
    mesh = plsc.VectorSubcoreMesh(core_axis_name="c", subcore_axis_name="s")
    out_type = (
        jax.ShapeDtypeStruct((2 * npad, HID), jnp.float32),
        jax.ShapeDtypeStruct((2 * npad, HID), jnp.float32),
    )
    pieces = []
    o = 0
    while o < rpt:
        ln = min(_CHUNK, rpt - o)
        pieces.append((o, ln))
        o += ln

    @functools.partial(
        pl.kernel, mesh=mesh, out_type=out_type,
        scratch_types=[
            pltpu.VMEM((_CHUNK,), jnp.int32),
            pltpu.VMEM((_CHUNK, HID), jnp.float32),
            pltpu.VMEM_SHARED((npad, HID), jnp.float32),
        ],
    )
    def k(m_hbm, a_hbm, row_hbm, z128_hbm, o128_hbm, oa_hbm,
          idxb, mbuf, acc128):
        c = lax.axis_index("c")
        s = lax.axis_index("s")
        wid = c * _NS + s

        def zero_acc():
            pltpu.sync_copy(z128_hbm, mbuf)
            for o_, ln in pieces:
                pltpu.sync_copy(mbuf.at[pl.ds(0, ln)],
                                acc128.at[pl.ds(s * rpt + o_, ln)])

        def drain_acc(dst):
            for o_, ln in pieces:
                pltpu.sync_copy(acc128.at[pl.ds(s * rpt + o_, ln)],
                                mbuf.at[pl.ds(0, ln)])
                pltpu.sync_copy(mbuf.at[pl.ds(0, ln)],
                                dst.at[pl.ds(c * npad + s * rpt + o_, ln)])

        def scat_loop(src_hbm):
            def body(j, carry):
                base = (wid * chunks + j) * _CHUNK
                sl = pl.ds(base, _CHUNK)
                pltpu.sync_copy(row_hbm.at[sl], idxb)
                pltpu.sync_copy(src_hbm.at[sl], mbuf)
                pltpu.sync_copy(mbuf, acc128.at[idxb], add=True)
                return carry
            lax.fori_loop(0, chunks, body, 0)

        zero_acc()
        plsc.subcore_barrier()
        scat_loop(m_hbm)
        plsc.subcore_barrier()
        drain_acc(o128_hbm)
        plsc.subcore_barrier()
        zero_acc()
        plsc.subcore_barrier()
        scat_loop(a_hbm)
        plsc.subcore_barrier()
        drain_acc(oa_hbm)

    return k(m, aux128, rowp, z128)


def _tc_edge(hr, hc, rels, w1a, w1b, w1c, w2, wc1, smalls, e_real, epad):
    BE = 1024
    nb = epad // BE

    def body(hr_ref, hc_ref, rl_ref, w1a_ref, w1b_ref, w1c_ref,
             w2_ref, wc1_ref, sm_ref, m_ref, aux_ref):
        sm = sm_ref[...]
        lane = jax.lax.broadcasted_iota(jnp.int32, (BE, 16), 1)
        rl = rl_ref[...]
        rel = jnp.where(lane < 3, rl, 0.0)
        d2 = jnp.sum(jnp.where(lane == 3, rl, 0.0), axis=1, keepdims=True)
        t = jnp.dot(hr_ref[...], w1a_ref[...], preferred_element_type=jnp.float32)
        t = t + jnp.dot(hc_ref[...], w1b_ref[...], preferred_element_type=jnp.float32)
        t = t + d2 * w1c_ref[...] + sm[0:1]
        m = _silu(t)
        t2 = jnp.dot(m, w2_ref[...], preferred_element_type=jnp.float32) + sm[1:2]
        m = _silu(t2)
        ba = jnp.sum(sm[3:4] * (1.0 / HID), axis=1, keepdims=True)
        att = _sig(jnp.sum(m * sm[2:3], axis=1, keepdims=True) + ba)
        m = m * att
        t3 = jnp.dot(m, wc1_ref[...], preferred_element_type=jnp.float32) + sm[4:5]
        cwh = _silu(t3)
        cw = jnp.sum(cwh * sm[5:6], axis=1, keepdims=True)
        rowid = (jax.lax.broadcasted_iota(jnp.int32, (BE, 1), 0)
                 + pl.program_id(0) * BE)
        msk = (rowid < e_real).astype(jnp.float32)
        m_ref[...] = m * msk
        aux = rel * cw
        aux = jnp.where(lane == 3, 1.0, aux)
        aux_ref[...] = jnp.concatenate(
            [aux * msk, jnp.zeros((BE, HID - 16), jnp.float32)], axis=1)

    const2 = lambda i: (0, 0)
    return pl.pallas_call(
        body,
        grid=(nb,),
        in_specs=[
            pl.BlockSpec((BE, HID), lambda i: (i, 0)),
            pl.BlockSpec((BE, HID), lambda i: (i, 0)),
            pl.BlockSpec((BE, 16), lambda i: (i, 0)),
            pl.BlockSpec((HID, HID), const2),
            pl.BlockSpec((HID, HID), const2),
            pl.BlockSpec((1, HID), const2),
            pl.BlockSpec((HID, HID), const2),
            pl.BlockSpec((HID, HID), const2),
            pl.BlockSpec((8, HID), const2),
        ],
        out_specs=[
            pl.BlockSpec((BE, HID), lambda i: (i, 0)),
            pl.BlockSpec((BE, HID), lambda i: (i, 0)),
        ],
        out_shape=[
            jax.ShapeDtypeStruct((epad, HID), jnp.float32),
            jax.ShapeDtypeStruct((epad, HID), jnp.float32),
        ],
    )(hr, hc, rels, w1a, w1b, w1c, w2, wc1, smalls)


def _tc_node(p0, p1, x0, x1, hidden, coords128, wn1a, wn1b, wn2, smalls, n):
    BN = 1000
    nb = n // BN

    def body(a0_ref, a1_ref, x0_ref, x1_ref, h_ref, c_ref,
             wa_ref, wb_ref, w2_ref, sm_ref, hpre_ref, oc_ref, st_ref, acc):
        i = pl.program_id(0)
        sm = sm_ref[...]
        hagg = a0_ref[...] + a1_ref[...]
        aux = x0_ref[...] + x1_ref[...]
        h = h_ref[...]
        t = (jnp.dot(h, wa_ref[...], preferred_element_type=jnp.float32)
             + jnp.dot(hagg, wb_ref[...], preferred_element_type=jnp.float32)
             + sm[0:1])
        hu = jnp.dot(_silu(t), w2_ref[...], preferred_element_type=jnp.float32) + sm[1:2]
        hpre = h + hu
        hpre_ref[...] = hpre

        @pl.when(i == 0)
        def _():
            acc[...] = jnp.zeros_like(acc)

        acc[0:1, :] = acc[0:1, :] + jnp.sum(hpre, axis=0, keepdims=True)
        acc[1:2, :] = acc[1:2, :] + jnp.sum(hpre * hpre, axis=0, keepdims=True)
        st_ref[...] = acc[...]

        lane = jax.lax.broadcasted_iota(jnp.int32, (BN, HID), 1)
        deg = jnp.sum(jnp.where(lane == 3, aux, 0.0), axis=1, keepdims=True)
        degc = jnp.maximum(deg, 1.0)
        cagg = jnp.where(lane < 3, aux, 0.0)
        oc_ref[...] = c_ref[...] + cagg / degc

    const2 = lambda i: (0, 0)
    return pl.pallas_call(
        body,
        grid=(nb,),
        in_specs=[
            pl.BlockSpec((BN, HID), lambda i: (i, 0)),
            pl.BlockSpec((BN, HID), lambda i: (i, 0)),
            pl.BlockSpec((BN, HID), lambda i: (i, 0)),
            pl.BlockSpec((BN, HID), lambda i: (i, 0)),
            pl.BlockSpec((BN, HID), lambda i: (i, 0)),
            pl.BlockSpec((BN, HID), lambda i: (i, 0)),
            pl.BlockSpec((HID, HID), const2),
            pl.BlockSpec((HID, HID), const2),
            pl.BlockSpec((HID, HID), const2),
            pl.BlockSpec((8, HID), const2),
        ],
        out_specs=[
            pl.BlockSpec((BN, HID), lambda i: (i, 0)),
            pl.BlockSpec((BN, HID), lambda i: (i, 0)),
            pl.BlockSpec((8, HID), const2),
        ],
        out_shape=[
            jax.ShapeDtypeStruct((n, HID), jnp.float32),
            jax.ShapeDtypeStruct((n, HID), jnp.float32),
            jax.ShapeDtypeStruct((8, HID), jnp.float32),
        ],
        scratch_shapes=[pltpu.VMEM((8, HID), jnp.float32)],
    )(p0, p1, x0, x1, hidden, coords128, wn1a, wn1b, wn2, smalls)


def _tc_norm(hpre, stats, n):
    BN = 1000
    nb = n // BN

    def body(h_ref, st_ref, o_ref):
        s = st_ref[...]
        mu = s[0:1, :] * (1.0 / n)
        var = (jnp.sum(s[1:2, :], axis=1, keepdims=True) * (1.0 / n)
               - jnp.sum(mu * mu, axis=1, keepdims=True))
        r = jax.lax.rsqrt(1e-6 + var)
        o_ref[...] = (h_ref[...] - mu) * r

    return pl.pallas_call(
        body,
        grid=(nb,),
        in_specs=[
            pl.BlockSpec((BN, HID), lambda i: (i, 0)),
            pl.BlockSpec((8, HID), lambda i: (0, 0)),
        ],
        out_specs=pl.BlockSpec((BN, HID), lambda i: (i, 0)),
        out_shape=jax.ShapeDtypeStruct((n, HID), jnp.float32),
    )(hpre, stats)


def kernel(batch_coords, batch_hidden, edges, We1, be1, We2, be2, Wa, ba,
           Wc1, bc1, Wc2, Wn1, bn1, Wn2, bn2):
    f32 = jnp.float32
    n = batch_hidden.shape[0]
    e = edges.shape[1]
    tile_e = _NW * _CHUNK
    epad = ((e + tile_e - 1) // tile_e) * tile_e

    coords128 = jnp.zeros((n, HID), f32).at[:, :3].set(batch_coords)
    coords4 = jnp.zeros((n, 4), f32).at[:, :3].set(batch_coords).reshape(-1)
    pad = epad - e
    rowp = jnp.concatenate([edges[0], jnp.zeros((pad,), jnp.int32)])
    colp = jnp.concatenate([edges[1], jnp.zeros((pad,), jnp.int32)])

    hr, hc, rels_flat = _sc_gather(batch_hidden, coords4, rowp, colp, n, epad)
    rels = rels_flat.reshape(epad, 16)

    w1a = We1[:HID]
    w1b = We1[HID:2 * HID]
    w1c = We1[2 * HID:]
    smalls = (jnp.zeros((8, HID), f32)
              .at[0].set(be1).at[1].set(be2).at[2].set(Wa[:, 0])
              .at[3].set(jnp.full((HID,), ba[0], f32))
              .at[4].set(bc1).at[5].set(Wc2[:, 0]))
    m, aux = _tc_edge(hr, hc, rels, w1a, w1b, w1c, We2, Wc1, smalls, e, epad)

    npad = ((n + 8 * _NS - 1) // (8 * _NS)) * (8 * _NS)
    z128 = jnp.zeros((_CHUNK, HID), f32)
    o128, oa = _sc_scatter(m, aux, rowp, z128, npad, epad)
    p0, p1 = o128[:n], o128[npad:npad + n]
    x0, x1 = oa[:n], oa[npad:npad + n]

    wn1a = Wn1[:HID]
    wn1b = Wn1[HID:]
    smalls2 = jnp.zeros((8, HID), f32).at[0].set(bn1).at[1].set(bn2)
    hpre, oc, stats = _tc_node(p0, p1, x0, x1, batch_hidden, coords128,
                               wn1a, wn1b, Wn2, smalls2, n)
    out_hidden = _tc_norm(hpre, stats, n)
    return (oc[:, :3], out_hidden)
